# baseline (speedup 1.0000x reference)
import functools

import jax
import jax.numpy as jnp
from jax import lax
from jax.experimental import pallas as pl
from jax.experimental.pallas import tpu as pltpu
from jax.experimental.pallas import tpu_sc as plsc

_B, _H, _W, _K = 8, 256, 256, 64
_BAND = 24
_LAMBDA_FLUX = 0.1


def _round_half_even_nonneg(x):
    t = x.astype(jnp.int32)
    r = x - t.astype(jnp.float32)
    inc = (r > 0.5) | ((r == 0.5) & ((t & 1) == 1))
    return t + jnp.where(inc, 1, 0)


def _tc_body(cent_ref, hm_ref, out_ref, hmt_ref):
    b = pl.program_id(0)
    hmt_ref[...] = jnp.zeros((_H, _W), jnp.float32)
    col_i = lax.broadcasted_iota(jnp.int32, (_BAND, _W), 1).astype(jnp.float32)
    row_i = lax.broadcasted_iota(jnp.int32, (_BAND, _W), 0).astype(jnp.float32)

    def gband(k):
        cx = cent_ref[b, 0, k] * jnp.float32(_W - 1)
        cy = cent_ref[b, 1, k] * jnp.float32(_H - 1)
        kxi = jnp.clip(_round_half_even_nonneg(cx), 0, _W - 1)
        kyi = jnp.clip(_round_half_even_nonneg(cy), 0, _H - 1)
        start = jnp.minimum((jnp.maximum(kyi - 7, 0) // 8) * 8, _H - _BAND)
        start = pl.multiple_of(start, 8)
        start_f = start.astype(jnp.float32)
        rowt = row_i + (start_f - cy)
        d2 = (col_i - cx) ** 2 + rowt * rowt
        g = jnp.exp(-d2 / 8.0)
        win = (jnp.abs(col_i - kxi.astype(jnp.float32)) <= 7.0) & (
            jnp.abs(row_i + (start_f - kyi.astype(jnp.float32))) <= 7.0)
        return jnp.where(win, g, 0.0), start

    def merge(g, start):
        start = pl.multiple_of(start, 8)
        hmt_ref[pl.ds(start, _BAND), :] = jnp.maximum(
            hmt_ref[pl.ds(start, _BAND), :], g)

    def body(k, carry):
        g_prev, start_prev = carry
        g_k, start_k = gband(k)
        merge(g_prev, start_prev)
        return g_k, start_k

    g_last, start_last = lax.fori_loop(
        0, _K, body, (jnp.zeros((_BAND, _W), jnp.float32), 0))
    merge(g_last, start_last)

    p = jnp.clip(hm_ref[0, 0], 1e-6, 1.0 - 1e-6)
    t = hmt_ref[...]
    posm = t == 1.0
    lp = jnp.log(jnp.where(posm, p, 1.0 - p))
    omt = 1.0 - t
    omt2 = omt * omt
    negv = omt2 * omt2 * p * p
    omp = 1.0 - p
    posv = omp * omp
    contrib = -jnp.sum(jnp.where(posm, posv, negv) * lp)
    npos_cnt = jnp.sum(posm.astype(jnp.float32))

    oi = lax.broadcasted_iota(jnp.int32, (1, 1, 8), 2)
    out_ref[...] = jnp.where(oi == 0, contrib,
                             jnp.where(oi == 1, npos_cnt, 0.0))


def _sc_body(cxs_hbm, cys_hbm, gfl_hbm, vals_hbm, out_hbm,
             cxb_v, cyb_v, mycx_v, mycy_v, gfl_v, keys_v, idx_v, gath_v,
             outv_v, sem):
    c = lax.axis_index("c")
    s = lax.axis_index("s")
    wid = s * 2 + c
    base = wid * 16
    b = wid // 4
    bbase = b * 64

    pltpu.sync_copy(cxs_hbm.at[pl.ds(bbase, 64)], cxb_v)
    pltpu.sync_copy(cys_hbm.at[pl.ds(bbase, 64)], cyb_v)
    pltpu.sync_copy(cxs_hbm.at[pl.ds(base, 16)], mycx_v)
    pltpu.sync_copy(cys_hbm.at[pl.ds(base, 16)], mycy_v)
    pltpu.sync_copy(gfl_hbm.at[pl.ds(base, 16)], gfl_v)

    iota = lax.iota(jnp.int32, 16)

    def cell(xv, lim):
        return jnp.clip(_round_half_even_nonneg(xv * jnp.float32(lim - 1)),
                        0, lim - 1)

    mycx = mycx_v[...] * jnp.float32(_W - 1)
    mycy = mycy_v[...] * jnp.float32(_H - 1)
    kx = cell(mycx_v[...], _W)
    ky = cell(mycy_v[...], _H)
    dx = mycx - kx.astype(jnp.float32)
    dy = mycy - ky.astype(jnp.float32)
    mykey = ky * _W + kx

    for d in range(4):
        kxd = cell(cxb_v[pl.ds(d * 16, 16)], _W)
        kyd = cell(cyb_v[pl.ds(d * 16, 16)], _H)
        keys_v[pl.ds(d * 16, 16)] = kyd * _W + kxd
    for d in range(4):
        keys_v[pl.ds(64 + d * 16, 16)] = iota * 0 - 1

    boff = (wid % 4) * 16

    def wbody(shift, loser):
        other = keys_v[pl.ds(boff + shift, 16)]
        return loser | (other == mykey)

    loser = lax.fori_loop(1, 64, wbody, iota < 0)
    w = jnp.where(loser, 0.0, 1.0)

    pix = ky * _W + kx
    ob = (2 * b) * (_H * _W)
    idx_v[pl.ds(0, 16)] = ob + pix
    idx_v[pl.ds(16, 16)] = ob + _H * _W + pix
    idx_v[pl.ds(32, 16)] = 2 * _B * _H * _W + b * (_H * _W) + pix
    pltpu.async_copy(vals_hbm.at[idx_v], gath_v, sem).wait()
    o0 = gath_v[pl.ds(0, 16)]
    o1 = gath_v[pl.ds(16, 16)]
    fl = gath_v[pl.ds(32, 16)]

    outv_v[pl.ds(0, 16)] = w * (jnp.abs(o0 - dx) + jnp.abs(o1 - dy))
    outv_v[pl.ds(16, 16)] = w * jnp.abs(fl - gfl_v[...])
    outv_v[pl.ds(32, 16)] = w
    pltpu.sync_copy(outv_v, out_hbm.at[wid])


def _tc_call(cent, heatmap):
    return pl.pallas_call(
        _tc_body,
        grid=(_B,),
        in_specs=[
            pl.BlockSpec(memory_space=pltpu.SMEM),
            pl.BlockSpec((1, 1, _H, _W), lambda b: (b, 0, 0, 0)),
        ],
        out_specs=pl.BlockSpec((1, 1, 8), lambda b: (b, 0, 0)),
        out_shape=jax.ShapeDtypeStruct((_B, 1, 8), jnp.float32),
        scratch_shapes=[pltpu.VMEM((_H, _W), jnp.float32)],
    )(cent, heatmap)


def _sc_call(cxs, cys, gfl, vals):
    mesh = plsc.VectorSubcoreMesh(core_axis_name="c", subcore_axis_name="s")
    f = functools.partial(
        pl.kernel,
        mesh=mesh,
        out_type=jax.ShapeDtypeStruct((32, 48), jnp.float32),
        scratch_types=[
            pltpu.VMEM((64,), jnp.float32),
            pltpu.VMEM((64,), jnp.float32),
            pltpu.VMEM((16,), jnp.float32),
            pltpu.VMEM((16,), jnp.float32),
            pltpu.VMEM((16,), jnp.float32),
            pltpu.VMEM((128,), jnp.int32),
            pltpu.VMEM((48,), jnp.int32),
            pltpu.VMEM((48,), jnp.float32),
            pltpu.VMEM((48,), jnp.float32),
            pltpu.SemaphoreType.DMA,
        ],
    )(_sc_body)
    return f(cxs, cys, gfl, vals)


def kernel(heatmap, offset, log_flux, gt_centroids, gt_log_flux):
    B, _, H, W = heatmap.shape
    K = gt_centroids.shape[1]

    cent = jnp.transpose(gt_centroids, (0, 2, 1))
    tc_out = _tc_call(cent, heatmap)

    vals = jnp.concatenate([offset.reshape(-1), log_flux.reshape(-1)])
    cxs = gt_centroids[:, :, 0].reshape(-1)
    cys = gt_centroids[:, :, 1].reshape(-1)
    gfl = gt_log_flux.reshape(-1)
    sc_out = _sc_call(cxs, cys, gfl, vals)

    nposf = jnp.maximum(tc_out[:, 0, 1].sum(), 1.0)
    l_hm = tc_out[:, 0, 0].sum() / nposf

    npos = jnp.maximum(sc_out[:, 32:48].sum(), 1.0)
    l_off = sc_out[:, 0:16].sum() / npos
    l_fl = _LAMBDA_FLUX * (sc_out[:, 16:32].sum() / npos)

    total = l_hm + l_off + l_fl
    return (l_hm, l_off, l_fl, total, jnp.float32(K))


# speedup vs baseline: 1.0432x; 1.0432x over previous
import functools

import jax
import jax.numpy as jnp
from jax import lax
from jax.experimental import pallas as pl
from jax.experimental.pallas import tpu as pltpu
from jax.experimental.pallas import tpu_sc as plsc

_B, _H, _W, _K = 8, 256, 256, 64
_BAND = 24
_LAMBDA_FLUX = 0.1


def _round_half_even_nonneg(x):
    t = x.astype(jnp.int32)
    r = x - t.astype(jnp.float32)
    inc = (r > 0.5) | ((r == 0.5) & ((t & 1) == 1))
    return t + jnp.where(inc, 1, 0)


def _tc_body(cent_ref, hm_ref, out_ref, hmt_ref):
    b = pl.program_id(0)
    hmt_ref[...] = jnp.zeros((_H, _W), jnp.float32)
    col_i = lax.broadcasted_iota(jnp.int32, (_BAND, _W), 1).astype(jnp.float32)
    row_i = lax.broadcasted_iota(jnp.int32, (_BAND, _W), 0).astype(jnp.float32)

    def gband(k):
        cx = cent_ref[b, 0, k] * jnp.float32(_W - 1)
        cy = cent_ref[b, 1, k] * jnp.float32(_H - 1)
        kxi = jnp.clip(_round_half_even_nonneg(cx), 0, _W - 1)
        kyi = jnp.clip(_round_half_even_nonneg(cy), 0, _H - 1)
        start = jnp.minimum((jnp.maximum(kyi - 7, 0) // 8) * 8, _H - _BAND)
        start = pl.multiple_of(start, 8)
        start_f = start.astype(jnp.float32)
        rowt = row_i + (start_f - cy)
        d2 = (col_i - cx) ** 2 + rowt * rowt
        g = jnp.exp(-d2 / 8.0)
        win = (jnp.abs(col_i - kxi.astype(jnp.float32)) <= 7.0) & (
            jnp.abs(row_i + (start_f - kyi.astype(jnp.float32))) <= 7.0)
        return jnp.where(win, g, 0.0), start

    def merge(g, start):
        start = pl.multiple_of(start, 8)
        hmt_ref[pl.ds(start, _BAND), :] = jnp.maximum(
            hmt_ref[pl.ds(start, _BAND), :], g)

    def body(k, carry):
        g_prev, start_prev = carry
        g_k, start_k = gband(k)
        merge(g_prev, start_prev)
        return g_k, start_k

    g_last, start_last = lax.fori_loop(
        0, _K, body, (jnp.zeros((_BAND, _W), jnp.float32), 0))
    merge(g_last, start_last)

    p = jnp.clip(hm_ref[0, 0], 1e-6, 1.0 - 1e-6)
    t = hmt_ref[...]
    posm = t == 1.0
    lp = jnp.log(jnp.where(posm, p, 1.0 - p))
    omt = 1.0 - t
    omt2 = omt * omt
    negv = omt2 * omt2 * p * p
    omp = 1.0 - p
    posv = omp * omp
    contrib = -jnp.sum(jnp.where(posm, posv, negv) * lp)
    npos_cnt = jnp.sum(posm.astype(jnp.float32))

    oi = lax.broadcasted_iota(jnp.int32, (1, 1, 8), 2)
    out_ref[...] = jnp.where(oi == 0, contrib,
                             jnp.where(oi == 1, npos_cnt, 0.0))


def _sc_body(stage_hbm, off_hbm, flux_hbm, out_hbm,
             buf_v, keys_v, idxo_v, idxf_v, gatho_v, gathf_v,
             outv_v, semo, semf):
    c = lax.axis_index("c")
    s = lax.axis_index("s")
    wid = s * 2 + c
    b = wid // 4
    boff = (wid % 4) * 16

    pltpu.sync_copy(stage_hbm.at[pl.ds(b * 192, 192)], buf_v)

    iota = lax.iota(jnp.int32, 16)

    def cell(xv, lim):
        return jnp.clip(_round_half_even_nonneg(xv * jnp.float32(lim - 1)),
                        0, lim - 1)

    mycx = buf_v[pl.ds(boff, 16)] * jnp.float32(_W - 1)
    mycy = buf_v[pl.ds(64 + boff, 16)] * jnp.float32(_H - 1)
    gfl = buf_v[pl.ds(128 + boff, 16)]
    kx = cell(buf_v[pl.ds(boff, 16)], _W)
    ky = cell(buf_v[pl.ds(64 + boff, 16)], _H)
    dx = mycx - kx.astype(jnp.float32)
    dy = mycy - ky.astype(jnp.float32)
    mykey = ky * _W + kx

    pix = ky * _W + kx
    ob = (2 * b) * (_H * _W)
    idxo_v[pl.ds(0, 16)] = ob + pix
    idxo_v[pl.ds(16, 16)] = ob + _H * _W + pix
    idxf_v[...] = b * (_H * _W) + pix
    cp_o = pltpu.async_copy(off_hbm.at[idxo_v], gatho_v, semo)
    cp_f = pltpu.async_copy(flux_hbm.at[idxf_v], gathf_v, semf)

    for d in range(4):
        kxd = cell(buf_v[pl.ds(d * 16, 16)], _W)
        kyd = cell(buf_v[pl.ds(64 + d * 16, 16)], _H)
        keys_v[pl.ds(d * 16, 16)] = kyd * _W + kxd
    for d in range(4):
        keys_v[pl.ds(64 + d * 16, 16)] = iota * 0 - 1

    def wbody(shift, loser):
        other = keys_v[pl.ds(boff + shift, 16)]
        return loser | (other == mykey)

    loser = lax.fori_loop(1, 64, wbody, iota < 0)
    w = jnp.where(loser, 0.0, 1.0)

    cp_o.wait()
    cp_f.wait()
    o0 = gatho_v[pl.ds(0, 16)]
    o1 = gatho_v[pl.ds(16, 16)]
    fl = gathf_v[...]

    outv_v[pl.ds(0, 16)] = w * (jnp.abs(o0 - dx) + jnp.abs(o1 - dy))
    outv_v[pl.ds(16, 16)] = w * jnp.abs(fl - gfl)
    outv_v[pl.ds(32, 16)] = w
    pltpu.sync_copy(outv_v, out_hbm.at[wid])


def _tc_call(cent, heatmap):
    return pl.pallas_call(
        _tc_body,
        grid=(_B,),
        in_specs=[
            pl.BlockSpec(memory_space=pltpu.SMEM),
            pl.BlockSpec((1, 1, _H, _W), lambda b: (b, 0, 0, 0)),
        ],
        out_specs=pl.BlockSpec((1, 1, 8), lambda b: (b, 0, 0)),
        out_shape=jax.ShapeDtypeStruct((_B, 1, 8), jnp.float32),
        scratch_shapes=[pltpu.VMEM((_H, _W), jnp.float32)],
    )(cent, heatmap)


def _sc_call(stage, off_flat, flux_flat):
    mesh = plsc.VectorSubcoreMesh(core_axis_name="c", subcore_axis_name="s")
    f = functools.partial(
        pl.kernel,
        mesh=mesh,
        out_type=jax.ShapeDtypeStruct((32, 48), jnp.float32),
        scratch_types=[
            pltpu.VMEM((192,), jnp.float32),
            pltpu.VMEM((128,), jnp.int32),
            pltpu.VMEM((32,), jnp.int32),
            pltpu.VMEM((16,), jnp.int32),
            pltpu.VMEM((32,), jnp.float32),
            pltpu.VMEM((16,), jnp.float32),
            pltpu.VMEM((48,), jnp.float32),
            pltpu.SemaphoreType.DMA,
            pltpu.SemaphoreType.DMA,
        ],
    )(_sc_body)
    return f(stage, off_flat, flux_flat)


def kernel(heatmap, offset, log_flux, gt_centroids, gt_log_flux):
    B, _, H, W = heatmap.shape
    K = gt_centroids.shape[1]

    cent = jnp.transpose(gt_centroids, (0, 2, 1))

    stage = jnp.concatenate(
        [gt_centroids[:, :, 0], gt_centroids[:, :, 1], gt_log_flux],
        axis=1).reshape(-1)
    sc_out = _sc_call(stage, offset.reshape(-1), log_flux.reshape(-1))
    tc_out = _tc_call(cent, heatmap)

    nposf = jnp.maximum(tc_out[:, 0, 1].sum(), 1.0)
    l_hm = tc_out[:, 0, 0].sum() / nposf

    npos = jnp.maximum(sc_out[:, 32:48].sum(), 1.0)
    l_off = sc_out[:, 0:16].sum() / npos
    l_fl = _LAMBDA_FLUX * (sc_out[:, 16:32].sum() / npos)

    total = l_hm + l_off + l_fl
    return (l_hm, l_off, l_fl, total, jnp.float32(K))


# speedup vs baseline: 1.1967x; 1.1471x over previous
import functools

import jax
import jax.numpy as jnp
from jax import lax
from jax.experimental import pallas as pl
from jax.experimental.pallas import tpu as pltpu
from jax.experimental.pallas import tpu_sc as plsc

_B, _H, _W, _K = 8, 256, 256, 64
_BAND = 24
_LAMBDA_FLUX = 0.1


def _round_half_even_nonneg(x):
    t = x.astype(jnp.int32)
    r = x - t.astype(jnp.float32)
    inc = (r > 0.5) | ((r == 0.5) & ((t & 1) == 1))
    return t + jnp.where(inc, 1, 0)


def _tc_body(cent_ref, hm_ref, out_ref, hmt0_ref, hmt1_ref, hmt2_ref,
             hmt3_ref):
    b = pl.program_id(0)
    bufs = (hmt0_ref, hmt1_ref, hmt2_ref, hmt3_ref)
    for buf in bufs:
        buf[...] = jnp.zeros((_H, _W), jnp.float32)
    col_i = lax.broadcasted_iota(jnp.int32, (_BAND, _W), 1).astype(jnp.float32)
    row_i = lax.broadcasted_iota(jnp.int32, (_BAND, _W), 0).astype(jnp.float32)

    def gband(k):
        cx = cent_ref[b, 0, k] * jnp.float32(_W - 1)
        cy = cent_ref[b, 1, k] * jnp.float32(_H - 1)
        kxi = jnp.clip(_round_half_even_nonneg(cx), 0, _W - 1)
        kyi = jnp.clip(_round_half_even_nonneg(cy), 0, _H - 1)
        start = jnp.minimum((jnp.maximum(kyi - 7, 0) // 8) * 8, _H - _BAND)
        start = pl.multiple_of(start, 8)
        start_f = start.astype(jnp.float32)
        rowt = row_i + (start_f - cy)
        d2 = (col_i - cx) ** 2 + rowt * rowt
        g = jnp.exp(-d2 / 8.0)
        win = (jnp.abs(col_i - kxi.astype(jnp.float32)) <= 7.0) & (
            jnp.abs(row_i + (start_f - kyi.astype(jnp.float32))) <= 7.0)
        return jnp.where(win, g, 0.0), start

    def merge(buf, g, start):
        start = pl.multiple_of(start, 8)
        buf[pl.ds(start, _BAND), :] = jnp.maximum(
            buf[pl.ds(start, _BAND), :], g)

    def body(i, carry):
        for j, buf in enumerate(bufs):
            g, start = gband(i * 4 + j)
            merge(buf, g, start)
        return carry

    lax.fori_loop(0, _K // 4, body, 0)

    p = jnp.clip(hm_ref[0, 0], 1e-6, 1.0 - 1e-6)
    t = jnp.maximum(jnp.maximum(hmt0_ref[...], hmt1_ref[...]),
                    jnp.maximum(hmt2_ref[...], hmt3_ref[...]))
    posm = t == 1.0
    lp = jnp.log(jnp.where(posm, p, 1.0 - p))
    omt = 1.0 - t
    omt2 = omt * omt
    negv = omt2 * omt2 * p * p
    omp = 1.0 - p
    posv = omp * omp
    contrib = -jnp.sum(jnp.where(posm, posv, negv) * lp)
    npos_cnt = jnp.sum(posm.astype(jnp.float32))

    oi = lax.broadcasted_iota(jnp.int32, (1, 1, 8), 2)
    out_ref[...] = jnp.where(oi == 0, contrib,
                             jnp.where(oi == 1, npos_cnt, 0.0))


def _sc_body(stage_hbm, off_hbm, flux_hbm, out_hbm,
             buf_v, keys_v, idxo_v, idxf_v, gatho_v, gathf_v,
             outv_v, semo, semf):
    c = lax.axis_index("c")
    s = lax.axis_index("s")
    wid = s * 2 + c
    b = wid // 4
    boff = (wid % 4) * 16

    pltpu.sync_copy(stage_hbm.at[pl.ds(b * 192, 192)], buf_v)

    iota = lax.iota(jnp.int32, 16)

    def cell(xv, lim):
        return jnp.clip(_round_half_even_nonneg(xv * jnp.float32(lim - 1)),
                        0, lim - 1)

    mycx = buf_v[pl.ds(boff, 16)] * jnp.float32(_W - 1)
    mycy = buf_v[pl.ds(64 + boff, 16)] * jnp.float32(_H - 1)
    gfl = buf_v[pl.ds(128 + boff, 16)]
    kx = cell(buf_v[pl.ds(boff, 16)], _W)
    ky = cell(buf_v[pl.ds(64 + boff, 16)], _H)
    dx = mycx - kx.astype(jnp.float32)
    dy = mycy - ky.astype(jnp.float32)
    mykey = ky * _W + kx

    pix = ky * _W + kx
    ob = (2 * b) * (_H * _W)
    idxo_v[pl.ds(0, 16)] = ob + pix
    idxo_v[pl.ds(16, 16)] = ob + _H * _W + pix
    idxf_v[...] = b * (_H * _W) + pix
    cp_o = pltpu.async_copy(off_hbm.at[idxo_v], gatho_v, semo)
    cp_f = pltpu.async_copy(flux_hbm.at[idxf_v], gathf_v, semf)

    for d in range(4):
        kxd = cell(buf_v[pl.ds(d * 16, 16)], _W)
        kyd = cell(buf_v[pl.ds(64 + d * 16, 16)], _H)
        keys_v[pl.ds(d * 16, 16)] = kyd * _W + kxd
    for d in range(4):
        keys_v[pl.ds(64 + d * 16, 16)] = iota * 0 - 1

    def wbody(shift, loser):
        other = keys_v[pl.ds(boff + shift, 16)]
        return loser | (other == mykey)

    loser = lax.fori_loop(1, 64, wbody, iota < 0)
    w = jnp.where(loser, 0.0, 1.0)

    cp_o.wait()
    cp_f.wait()
    o0 = gatho_v[pl.ds(0, 16)]
    o1 = gatho_v[pl.ds(16, 16)]
    fl = gathf_v[...]

    outv_v[pl.ds(0, 16)] = w * (jnp.abs(o0 - dx) + jnp.abs(o1 - dy))
    outv_v[pl.ds(16, 16)] = w * jnp.abs(fl - gfl)
    outv_v[pl.ds(32, 16)] = w
    pltpu.sync_copy(outv_v, out_hbm.at[wid])


def _tc_call(cent, heatmap):
    return pl.pallas_call(
        _tc_body,
        grid=(_B,),
        in_specs=[
            pl.BlockSpec(memory_space=pltpu.SMEM),
            pl.BlockSpec((1, 1, _H, _W), lambda b: (b, 0, 0, 0)),
        ],
        out_specs=pl.BlockSpec((1, 1, 8), lambda b: (b, 0, 0)),
        out_shape=jax.ShapeDtypeStruct((_B, 1, 8), jnp.float32),
        scratch_shapes=[pltpu.VMEM((_H, _W), jnp.float32)] * 4,
    )(cent, heatmap)


def _sc_call(stage, off_flat, flux_flat):
    mesh = plsc.VectorSubcoreMesh(core_axis_name="c", subcore_axis_name="s")
    f = functools.partial(
        pl.kernel,
        mesh=mesh,
        out_type=jax.ShapeDtypeStruct((32, 48), jnp.float32),
        scratch_types=[
            pltpu.VMEM((192,), jnp.float32),
            pltpu.VMEM((128,), jnp.int32),
            pltpu.VMEM((32,), jnp.int32),
            pltpu.VMEM((16,), jnp.int32),
            pltpu.VMEM((32,), jnp.float32),
            pltpu.VMEM((16,), jnp.float32),
            pltpu.VMEM((48,), jnp.float32),
            pltpu.SemaphoreType.DMA,
            pltpu.SemaphoreType.DMA,
        ],
    )(_sc_body)
    return f(stage, off_flat, flux_flat)


def kernel(heatmap, offset, log_flux, gt_centroids, gt_log_flux):
    B, _, H, W = heatmap.shape
    K = gt_centroids.shape[1]

    cent = jnp.transpose(gt_centroids, (0, 2, 1))

    stage = jnp.concatenate(
        [gt_centroids[:, :, 0], gt_centroids[:, :, 1], gt_log_flux],
        axis=1).reshape(-1)
    sc_out = _sc_call(stage, offset.reshape(-1), log_flux.reshape(-1))
    tc_out = _tc_call(cent, heatmap)

    nposf = jnp.maximum(tc_out[:, 0, 1].sum(), 1.0)
    l_hm = tc_out[:, 0, 0].sum() / nposf

    npos = jnp.maximum(sc_out[:, 32:48].sum(), 1.0)
    l_off = sc_out[:, 0:16].sum() / npos
    l_fl = _LAMBDA_FLUX * (sc_out[:, 16:32].sum() / npos)

    total = l_hm + l_off + l_fl
    return (l_hm, l_off, l_fl, total, jnp.float32(K))


# speedup vs baseline: 1.1971x; 1.0003x over previous
import functools

import jax
import jax.numpy as jnp
from jax import lax
from jax.experimental import pallas as pl
from jax.experimental.pallas import tpu as pltpu
from jax.experimental.pallas import tpu_sc as plsc

_B, _H, _W, _K = 8, 256, 256, 64
_BAND = 24
_LAMBDA_FLUX = 0.1


def _round_half_even_nonneg(x):
    t = x.astype(jnp.int32)
    r = x - t.astype(jnp.float32)
    inc = (r > 0.5) | ((r == 0.5) & ((t & 1) == 1))
    return t + jnp.where(inc, 1, 0)


def _tc_body(cent_ref, hm_ref, out_ref, *hmt_refs):
    b = pl.program_id(0)
    bufs = hmt_refs
    for buf in bufs:
        buf[...] = jnp.zeros((_H, _W), jnp.float32)
    col_i = lax.broadcasted_iota(jnp.int32, (_BAND, _W), 1).astype(jnp.float32)
    row_i = lax.broadcasted_iota(jnp.int32, (_BAND, _W), 0).astype(jnp.float32)

    def gband(k):
        cx = cent_ref[b, 0, k] * jnp.float32(_W - 1)
        cy = cent_ref[b, 1, k] * jnp.float32(_H - 1)
        kxi = jnp.clip(_round_half_even_nonneg(cx), 0, _W - 1)
        kyi = jnp.clip(_round_half_even_nonneg(cy), 0, _H - 1)
        start = jnp.minimum((jnp.maximum(kyi - 7, 0) // 8) * 8, _H - _BAND)
        start = pl.multiple_of(start, 8)
        start_f = start.astype(jnp.float32)
        rowt = row_i + (start_f - cy)
        d2 = (col_i - cx) ** 2 + rowt * rowt
        g = jnp.exp(-d2 / 8.0)
        win = (jnp.abs(col_i - kxi.astype(jnp.float32)) <= 7.0) & (
            jnp.abs(row_i + (start_f - kyi.astype(jnp.float32))) <= 7.0)
        return jnp.where(win, g, 0.0), start

    def merge(buf, g, start):
        start = pl.multiple_of(start, 8)
        buf[pl.ds(start, _BAND), :] = jnp.maximum(
            buf[pl.ds(start, _BAND), :], g)

    nb = len(bufs)

    def body(i, carry):
        for j, buf in enumerate(bufs):
            g, start = gband(i * nb + j)
            merge(buf, g, start)
        return carry

    lax.fori_loop(0, _K // nb, body, 0)

    p = jnp.clip(hm_ref[0, 0], 1e-6, 1.0 - 1e-6)
    t = hmt_refs[0][...]
    for buf in hmt_refs[1:]:
        t = jnp.maximum(t, buf[...])
    posm = t == 1.0
    lp = jnp.log(jnp.where(posm, p, 1.0 - p))
    omt = 1.0 - t
    omt2 = omt * omt
    negv = omt2 * omt2 * p * p
    omp = 1.0 - p
    posv = omp * omp
    contrib = -jnp.sum(jnp.where(posm, posv, negv) * lp)
    npos_cnt = jnp.sum(posm.astype(jnp.float32))

    oi = lax.broadcasted_iota(jnp.int32, (1, 1, 8), 2)
    out_ref[...] = jnp.where(oi == 0, contrib,
                             jnp.where(oi == 1, npos_cnt, 0.0))


def _sc_body(stage_hbm, off_hbm, flux_hbm, out_hbm,
             buf_v, keys_v, idxo_v, idxf_v, gatho_v, gathf_v,
             outv_v, semo, semf):
    c = lax.axis_index("c")
    s = lax.axis_index("s")
    wid = s * 2 + c
    b = wid // 4
    boff = (wid % 4) * 16

    pltpu.sync_copy(stage_hbm.at[pl.ds(b * 192, 192)], buf_v)

    iota = lax.iota(jnp.int32, 16)

    def cell(xv, lim):
        return jnp.clip(_round_half_even_nonneg(xv * jnp.float32(lim - 1)),
                        0, lim - 1)

    mycx = buf_v[pl.ds(boff, 16)] * jnp.float32(_W - 1)
    mycy = buf_v[pl.ds(64 + boff, 16)] * jnp.float32(_H - 1)
    gfl = buf_v[pl.ds(128 + boff, 16)]
    kx = cell(buf_v[pl.ds(boff, 16)], _W)
    ky = cell(buf_v[pl.ds(64 + boff, 16)], _H)
    dx = mycx - kx.astype(jnp.float32)
    dy = mycy - ky.astype(jnp.float32)
    mykey = ky * _W + kx

    pix = ky * _W + kx
    ob = (2 * b) * (_H * _W)
    idxo_v[pl.ds(0, 16)] = ob + pix
    idxo_v[pl.ds(16, 16)] = ob + _H * _W + pix
    idxf_v[...] = b * (_H * _W) + pix
    cp_o = pltpu.async_copy(off_hbm.at[idxo_v], gatho_v, semo)
    cp_f = pltpu.async_copy(flux_hbm.at[idxf_v], gathf_v, semf)

    for d in range(4):
        kxd = cell(buf_v[pl.ds(d * 16, 16)], _W)
        kyd = cell(buf_v[pl.ds(64 + d * 16, 16)], _H)
        keys_v[pl.ds(d * 16, 16)] = kyd * _W + kxd
    for d in range(4):
        keys_v[pl.ds(64 + d * 16, 16)] = iota * 0 - 1

    def wbody(shift, loser):
        other = keys_v[pl.ds(boff + shift, 16)]
        return loser | (other == mykey)

    loser = lax.fori_loop(1, 64, wbody, iota < 0)
    w = jnp.where(loser, 0.0, 1.0)

    cp_o.wait()
    cp_f.wait()
    o0 = gatho_v[pl.ds(0, 16)]
    o1 = gatho_v[pl.ds(16, 16)]
    fl = gathf_v[...]

    outv_v[pl.ds(0, 16)] = w * (jnp.abs(o0 - dx) + jnp.abs(o1 - dy))
    outv_v[pl.ds(16, 16)] = w * jnp.abs(fl - gfl)
    outv_v[pl.ds(32, 16)] = w
    pltpu.sync_copy(outv_v, out_hbm.at[wid])


def _tc_call(cent, heatmap):
    return pl.pallas_call(
        _tc_body,
        grid=(_B,),
        in_specs=[
            pl.BlockSpec(memory_space=pltpu.SMEM),
            pl.BlockSpec((1, 1, _H, _W), lambda b: (b, 0, 0, 0)),
        ],
        out_specs=pl.BlockSpec((1, 1, 8), lambda b: (b, 0, 0)),
        out_shape=jax.ShapeDtypeStruct((_B, 1, 8), jnp.float32),
        scratch_shapes=[pltpu.VMEM((_H, _W), jnp.float32)] * 8,
    )(cent, heatmap)


def _sc_call(stage, off_flat, flux_flat):
    mesh = plsc.VectorSubcoreMesh(core_axis_name="c", subcore_axis_name="s")
    f = functools.partial(
        pl.kernel,
        mesh=mesh,
        out_type=jax.ShapeDtypeStruct((32, 48), jnp.float32),
        scratch_types=[
            pltpu.VMEM((192,), jnp.float32),
            pltpu.VMEM((128,), jnp.int32),
            pltpu.VMEM((32,), jnp.int32),
            pltpu.VMEM((16,), jnp.int32),
            pltpu.VMEM((32,), jnp.float32),
            pltpu.VMEM((16,), jnp.float32),
            pltpu.VMEM((48,), jnp.float32),
            pltpu.SemaphoreType.DMA,
            pltpu.SemaphoreType.DMA,
        ],
    )(_sc_body)
    return f(stage, off_flat, flux_flat)


def kernel(heatmap, offset, log_flux, gt_centroids, gt_log_flux):
    B, _, H, W = heatmap.shape
    K = gt_centroids.shape[1]

    cent = jnp.transpose(gt_centroids, (0, 2, 1))

    stage = jnp.concatenate(
        [gt_centroids[:, :, 0], gt_centroids[:, :, 1], gt_log_flux],
        axis=1).reshape(-1)
    sc_out = _sc_call(stage, offset.reshape(-1), log_flux.reshape(-1))
    tc_out = _tc_call(cent, heatmap)

    nposf = jnp.maximum(tc_out[:, 0, 1].sum(), 1.0)
    l_hm = tc_out[:, 0, 0].sum() / nposf

    npos = jnp.maximum(sc_out[:, 32:48].sum(), 1.0)
    l_off = sc_out[:, 0:16].sum() / npos
    l_fl = _LAMBDA_FLUX * (sc_out[:, 16:32].sum() / npos)

    total = l_hm + l_off + l_fl
    return (l_hm, l_off, l_fl, total, jnp.float32(K))
